# baseline (speedup 1.0000x reference)
import functools

import numpy as np
import jax
import jax.numpy as jnp
from jax import lax
from jax.experimental import pallas as pl
from jax.experimental.pallas import tpu as pltpu
from jax.experimental.pallas import tpu_sc as plsc

_OH, _OW, _G = 7, 7, 2
_SCALE = 0.25
_B, _C, _H, _W = 2, 128, 200, 200
_N = 1024

_PTS = _OH * _OW * _G * _G
_PPTS = 224
_NCH = _PPTS // 16
_ROWS = 2 * _PTS
_TBL = _B * _H * _W + 208
_CW = _C
_NC, _NS = 2, 16
_NWORK = _NC * _NS
_RPW = _N // _NWORK
_OUTF = _C * _OH * _OW


def _point_consts():
    ay = np.zeros(_PPTS, np.float32)
    ax = np.zeros(_PPTS, np.float32)
    for p in range(_PTS):
        b, s = divmod(p, _G * _G)
        oh, ow = divmod(b, _OW)
        gy, gx = divmod(s, _G)
        ay[p] = (oh + (gy + 0.5) / _G) / _OH - 0.5
        ax[p] = (ow + (gx + 0.5) / _G) / _OW - 0.5
    ay[_PTS:] = ay[_PTS - 1]
    ax[_PTS:] = ax[_PTS - 1]
    return ay, ax

_AY_CONST, _AX_CONST = _point_consts()


def _sc_body(tbl_hbm, par_hbm, ay_hbm, ax_hbm, out_hbm,
             par_v, ay_v, ax_v, idx_v, w_v, g_v, out_v, sem):
    wid = lax.axis_index("s") * _NC + lax.axis_index("c")
    pltpu.sync_copy(ay_hbm, ay_v)
    pltpu.sync_copy(ax_hbm, ax_v)
    pltpu.sync_copy(par_hbm.at[pl.ds(wid * (_RPW * 8), _RPW * 8)], par_v)

    lanes = lax.iota(jnp.int32, 16)

    def splat_par(j, k):
        return plsc.load_gather(par_v, [jnp.full((16,), j * 8 + k, jnp.int32)])

    def roi_body(j, _):
        cx = splat_par(j, 0)
        cy = splat_par(j, 1)
        rw = splat_par(j, 2)
        rh = splat_par(j, 3)
        cs = splat_par(j, 4)
        sn = splat_par(j, 5)
        base = splat_par(j, 6).astype(jnp.int32)

        for c in range(_NCH):
            ay = ay_v[pl.ds(c * 16, 16)]
            ax = ax_v[pl.ds(c * 16, 16)]
            yy = rh * ay
            xx = rw * ax
            x = xx * cs - yy * sn + cx
            y = xx * sn + yy * cs + cy
            valid = ((y > -1.0) & (y < float(_H))
                     & (x > -1.0) & (x < float(_W)))
            xc = jnp.minimum(jnp.maximum(x, 0.0), float(_W - 1))
            yc = jnp.minimum(jnp.maximum(y, 0.0), float(_H - 1))
            x0 = xc.astype(jnp.int32)
            y0 = yc.astype(jnp.int32)
            lx = xc - x0.astype(jnp.float32)
            ly = yc - y0.astype(jnp.float32)
            hx = 1.0 - lx
            hy = 1.0 - ly
            vm = jnp.where(valid, 0.25, 0.0)
            r00 = base + y0 * _W + x0
            half = c // 7
            col = (c % 7) * 16
            ws = (hy * hx * vm, hy * lx * vm, ly * hx * vm, ly * lx * vm)
            rs = (r00, r00 + _W)
            for k in range(2):
                idx_v[2 * k + half, pl.ds(col, 16)] = rs[k]
            for k in range(4):
                w_v[pl.ds(k * _PPTS + c * 16, 16)] = ws[k]

        copies = []
        for k in range(2):
            copies.append(pltpu.async_copy(
                tbl_hbm.at[idx_v.at[2 * k]],
                g_v.at[pl.ds(k * _PTS, 112)], sem))
            copies.append(pltpu.async_copy(
                tbl_hbm.at[idx_v.at[2 * k + 1, pl.ds(0, 84)]],
                g_v.at[pl.ds(k * _PTS + 112, 84)], sem))
        for cp in copies:
            cp.wait()

        mask_hi = jnp.int32(-65536)

        def bin_body(b, _):
            accs = None
            for s in range(4):
                for k in range(2):
                    r = k * _PTS + 4 * b + s
                    wx0 = plsc.load_gather(
                        w_v, [jnp.full((16,), 2 * k * _PPTS + 4 * b + s,
                                       jnp.int32)])
                    wx1 = plsc.load_gather(
                        w_v, [jnp.full((16,), (2 * k + 1) * _PPTS + 4 * b + s,
                                       jnp.int32)])
                    terms = []
                    for g8 in range(8):
                        pk = g_v[r, pl.ds(g8 * 16, 16)]
                        w = wx0 if g8 < 4 else wx1
                        ev = lax.bitcast_convert_type(
                            pk << 16, jnp.float32)
                        od = lax.bitcast_convert_type(
                            pk & mask_hi, jnp.float32)
                        terms.append(w * ev)
                        terms.append(w * od)
                    if accs is None:
                        accs = terms
                    else:
                        accs = [a + t for a, t in zip(accs, terms)]
            accs = [accs[i] + accs[i + 8] for i in range(8)]
            for g4 in range(4):
                for par in range(2):
                    sidx = (lanes * 2 + g4 * 32 + par) * (_OH * _OW) + b
                    plsc.store_scatter(out_v, [sidx], accs[g4 * 2 + par])
            return _

        lax.fori_loop(0, _OH * _OW, bin_body, None)
        pltpu.sync_copy(out_v, out_hbm.at[wid * _RPW + j])
        return _

    lax.fori_loop(0, _RPW, roi_body, None)


@jax.jit
def _roi_align_sc(tbl, params, ayc, axc):
    mesh = plsc.VectorSubcoreMesh(core_axis_name="c", subcore_axis_name="s")
    f = functools.partial(
        pl.kernel,
        out_type=jax.ShapeDtypeStruct((_N, _OUTF), jnp.float32),
        mesh=mesh,
        compiler_params=pltpu.CompilerParams(needs_layout_passes=False),
        scratch_types=[
            pltpu.VMEM((_RPW * 8,), jnp.float32),
            pltpu.VMEM((_PPTS,), jnp.float32),
            pltpu.VMEM((_PPTS,), jnp.float32),
            pltpu.VMEM((4, 112), jnp.int32),
            pltpu.VMEM((4 * _PPTS,), jnp.float32),
            pltpu.VMEM((_ROWS, _CW), jnp.int32),
            pltpu.VMEM((_OUTF,), jnp.float32),
            pltpu.SemaphoreType.DMA,
        ],
    )(_sc_body)
    return f(tbl, params, ayc, axc)


def kernel(inputs, rois):
    tbl = jnp.transpose(inputs, (0, 2, 3, 1)).reshape(_B * _H * _W, _C)
    tbl = jnp.concatenate(
        [tbl.astype(jnp.bfloat16),
         jnp.zeros((_TBL + 1 - _B * _H * _W, _C), jnp.bfloat16)], axis=0)
    pk = lax.bitcast_convert_type(
        tbl.reshape(_TBL + 1, _C // 2, 2), jnp.int32)
    tbl = jnp.concatenate([pk[:_TBL], pk[1:_TBL + 1]], axis=1)
    cx = rois[:, 1] * _SCALE
    cy = rois[:, 2] * _SCALE
    rw = jnp.maximum(rois[:, 3] * _SCALE, 1.0)
    rh = jnp.maximum(rois[:, 4] * _SCALE, 1.0)
    cs = jnp.cos(rois[:, 5])
    sn = jnp.sin(rois[:, 5])
    base = rois[:, 0] * float(_H * _W)
    zero = jnp.zeros_like(cx)
    params = jnp.stack([cx, cy, rw, rh, cs, sn, base, zero], 1).reshape(-1)
    out = _roi_align_sc(tbl, params,
                        jnp.asarray(_AY_CONST), jnp.asarray(_AX_CONST))
    return out.reshape(_N, _C, _OH, _OW)


# speedup vs baseline: 1.0794x; 1.0794x over previous
import functools

import numpy as np
import jax
import jax.numpy as jnp
from jax import lax
from jax.experimental import pallas as pl
from jax.experimental.pallas import tpu as pltpu
from jax.experimental.pallas import tpu_sc as plsc

_OH, _OW, _G = 7, 7, 2
_SCALE = 0.25
_B, _C, _H, _W = 2, 128, 200, 200
_N = 1024

_PTS = _OH * _OW * _G * _G
_PPTS = 224
_NCH = _PPTS // 16
_GPTS = 208
_ROWS = 4 * _GPTS
_TBL = _B * _H * _W + 208
_CW = _C // 2
_NC, _NS = 2, 16
_NWORK = _NC * _NS
_RPW = _N // _NWORK
_OUTF = _C * _OH * _OW


def _point_consts():
    ay = np.zeros(_PPTS, np.float32)
    ax = np.zeros(_PPTS, np.float32)
    for p in range(_PTS):
        b, s = divmod(p, _G * _G)
        oh, ow = divmod(b, _OW)
        gy, gx = divmod(s, _G)
        ay[p] = (oh + (gy + 0.5) / _G) / _OH - 0.5
        ax[p] = (ow + (gx + 0.5) / _G) / _OW - 0.5
    ay[_PTS:] = ay[_PTS - 1]
    ax[_PTS:] = ax[_PTS - 1]
    return ay, ax

_AY_CONST, _AX_CONST = _point_consts()


def _sc_body(tbl_hbm, par_hbm, ay_hbm, ax_hbm, out_hbm,
             par_v, ay_v, ax_v, idx_v, w_v, g_v, out_v, sem):
    wid = lax.axis_index("s") * _NC + lax.axis_index("c")
    pltpu.sync_copy(ay_hbm, ay_v)
    pltpu.sync_copy(ax_hbm, ax_v)
    pltpu.sync_copy(par_hbm.at[pl.ds(wid * (_RPW * 8), _RPW * 8)], par_v)

    lanes = lax.iota(jnp.int32, 16)

    def splat_par(j, k):
        return plsc.load_gather(par_v, [jnp.full((16,), j * 8 + k, jnp.int32)])

    def roi_body(j, _):
        cx = splat_par(j, 0)
        cy = splat_par(j, 1)
        rw = splat_par(j, 2)
        rh = splat_par(j, 3)
        cs = splat_par(j, 4)
        sn = splat_par(j, 5)
        base = splat_par(j, 6).astype(jnp.int32)

        for c in range(_NCH):
            ay = ay_v[pl.ds(c * 16, 16)]
            ax = ax_v[pl.ds(c * 16, 16)]
            yy = rh * ay
            xx = rw * ax
            x = xx * cs - yy * sn + cx
            y = xx * sn + yy * cs + cy
            valid = ((y > -1.0) & (y < float(_H))
                     & (x > -1.0) & (x < float(_W)))
            xc = jnp.minimum(jnp.maximum(x, 0.0), float(_W - 1))
            yc = jnp.minimum(jnp.maximum(y, 0.0), float(_H - 1))
            x0 = xc.astype(jnp.int32)
            y0 = yc.astype(jnp.int32)
            lx = xc - x0.astype(jnp.float32)
            ly = yc - y0.astype(jnp.float32)
            hx = 1.0 - lx
            hy = 1.0 - ly
            vm = jnp.where(valid, 0.25, 0.0)
            r00 = base + y0 * _W + x0
            half = c // 7
            col = (c % 7) * 16
            ws = (hy * hx * vm, hy * lx * vm, ly * hx * vm, ly * lx * vm)
            rs = (r00, r00 + 1, r00 + _W, r00 + _W + 1)
            for k in range(4):
                idx_v[2 * k + half, pl.ds(col, 16)] = rs[k]
                w_v[pl.ds(k * _PPTS + c * 16, 16)] = ws[k]

        copies = []
        for k in range(4):
            copies.append(pltpu.async_copy(
                tbl_hbm.at[idx_v.at[2 * k]],
                g_v.at[pl.ds(k * _GPTS, 112)], sem))
            copies.append(pltpu.async_copy(
                tbl_hbm.at[idx_v.at[2 * k + 1, pl.ds(0, 96)]],
                g_v.at[pl.ds(k * _GPTS + 112, 96)], sem))
        for cp in copies:
            cp.wait()

        def bin_body(b, _):
            accs = None
            for s in range(4):
                for k in range(4):
                    r = k * _GPTS + 4 * b + s
                    wi = k * _PPTS + 4 * b + s
                    wspl = plsc.load_gather(
                        w_v, [jnp.full((16,), wi, jnp.int32)])
                    terms = []
                    for g4 in range(4):
                        pk = g_v[r, pl.ds(g4 * 16, 16)]
                        ev = lax.bitcast_convert_type(pk << 16, jnp.float32)
                        od = lax.bitcast_convert_type(pk, jnp.float32)
                        terms.append(wspl * ev)
                        terms.append(wspl * od)
                    if accs is None:
                        accs = terms
                    else:
                        accs = [a + t for a, t in zip(accs, terms)]
            for g4 in range(4):
                for par in range(2):
                    sidx = (lanes * 2 + g4 * 32 + par) * (_OH * _OW) + b
                    plsc.store_scatter(out_v, [sidx], accs[g4 * 2 + par])
            return _

        lax.fori_loop(0, _OH * _OW, bin_body, None)
        pltpu.sync_copy(out_v, out_hbm.at[wid * _RPW + j])
        return _

    lax.fori_loop(0, _RPW, roi_body, None)


@jax.jit
def _roi_align_sc(tbl, params, ayc, axc):
    mesh = plsc.VectorSubcoreMesh(core_axis_name="c", subcore_axis_name="s")
    f = functools.partial(
        pl.kernel,
        out_type=jax.ShapeDtypeStruct((_N, _OUTF), jnp.float32),
        mesh=mesh,
        compiler_params=pltpu.CompilerParams(needs_layout_passes=False,
                                             use_tc_tiling_on_sc=False),
        scratch_types=[
            pltpu.VMEM((_RPW * 8,), jnp.float32),
            pltpu.VMEM((_PPTS,), jnp.float32),
            pltpu.VMEM((_PPTS,), jnp.float32),
            pltpu.VMEM((8, 112), jnp.int32),
            pltpu.VMEM((4 * _PPTS,), jnp.float32),
            pltpu.VMEM((_ROWS, _CW), jnp.int32),
            pltpu.VMEM((_OUTF,), jnp.float32),
            pltpu.SemaphoreType.DMA,
        ],
    )(_sc_body)
    return f(tbl, params, ayc, axc)


def kernel(inputs, rois):
    tbl = jnp.transpose(inputs, (0, 2, 3, 1)).reshape(_B * _H * _W, _C)
    tbl = jnp.concatenate(
        [tbl.astype(jnp.bfloat16),
         jnp.zeros((_TBL - _B * _H * _W, _C), jnp.bfloat16)], axis=0)
    tbl = lax.bitcast_convert_type(tbl.reshape(_TBL, _CW, 2), jnp.int32)
    cx = rois[:, 1] * _SCALE
    cy = rois[:, 2] * _SCALE
    rw = jnp.maximum(rois[:, 3] * _SCALE, 1.0)
    rh = jnp.maximum(rois[:, 4] * _SCALE, 1.0)
    cs = jnp.cos(rois[:, 5])
    sn = jnp.sin(rois[:, 5])
    base = rois[:, 0] * float(_H * _W)
    zero = jnp.zeros_like(cx)
    params = jnp.stack([cx, cy, rw, rh, cs, sn, base, zero], 1).reshape(-1)
    out = _roi_align_sc(tbl, params,
                        jnp.asarray(_AY_CONST), jnp.asarray(_AX_CONST))
    return out.reshape(_N, _C, _OH, _OW)


# speedup vs baseline: 1.8039x; 1.6712x over previous
import functools

import numpy as np
import jax
import jax.numpy as jnp
from jax import lax
from jax.experimental import pallas as pl
from jax.experimental.pallas import tpu as pltpu
from jax.experimental.pallas import tpu_sc as plsc

_OH, _OW, _G = 7, 7, 2
_SCALE = 0.25
_B, _C, _H, _W = 2, 128, 200, 200
_N = 1024

_PTS = _OH * _OW * _G * _G
_PPTS = 224
_NCH = _PPTS // 16
_ROWS = 2 * _PTS
_NC, _NS = 2, 16
_NWORK = _NC * _NS
_RPW = _N // _NWORK
_OUTF = _C * _OH * _OW
_CW = _C // 2

_PCHUNK = 128
_CPW = 20
_TBL = _NWORK * _CPW * _PCHUNK
_TBLF = _TBL + 8


def _point_consts():
    ay = np.zeros(_PPTS, np.float32)
    ax = np.zeros(_PPTS, np.float32)
    for p in range(_PTS):
        b, s = divmod(p, _G * _G)
        oh, ow = divmod(b, _OW)
        gy, gx = divmod(s, _G)
        ay[p] = (oh + (gy + 0.5) / _G) / _OH - 0.5
        ax[p] = (ow + (gx + 0.5) / _G) / _OW - 0.5
    ay[_PTS:] = ay[_PTS - 1]
    ax[_PTS:] = ax[_PTS - 1]
    return ay, ax

_AY_CONST, _AX_CONST = _point_consts()


def _pack_body(src_hbm, pair_hbm, in_v, out_v, sem):
    wid = lax.axis_index("s") * _NC + lax.axis_index("c")
    base_row = wid * (_CPW * _PCHUNK)

    def chunk_body(ci, _):
        a = base_row + ci * _PCHUNK
        pltpu.sync_copy(src_hbm.at[pl.ds(a, 136)], in_v)

        def pix_body(jj, _):
            for g in range(4):
                lo = lax.bitcast_convert_type(
                    in_v[jj, pl.ds(g * 32, 16)], jnp.int32)
                hi = lax.bitcast_convert_type(
                    in_v[jj, pl.ds(g * 32 + 16, 16)], jnp.int32)
                w = (lax.shift_right_logical(lo + 32768, 16)
                     | ((hi + 32768) & jnp.int32(-65536)))

                @pl.when(jj < _PCHUNK)
                def _store_lo():
                    out_v[jj, pl.ds(g * 16, 16)] = w

                @pl.when(jj > 0)
                def _store_hi():
                    out_v[jj - 1, pl.ds(_CW + g * 16, 16)] = w
            return _

        lax.fori_loop(0, _PCHUNK + 1, pix_body, None)
        pltpu.sync_copy(out_v, pair_hbm.at[pl.ds(a, _PCHUNK)])
        return _

    lax.fori_loop(0, _CPW, chunk_body, None)


def _sc_body(tbl_hbm, par_hbm, ay_hbm, ax_hbm, out_hbm,
             par_v, ay_v, ax_v, idx_v, w_v, g_v, out_v, sem):
    wid = lax.axis_index("s") * _NC + lax.axis_index("c")
    pltpu.sync_copy(ay_hbm, ay_v)
    pltpu.sync_copy(ax_hbm, ax_v)
    pltpu.sync_copy(par_hbm.at[pl.ds(wid * (_RPW * 8), _RPW * 8)], par_v)

    lanes = lax.iota(jnp.int32, 16)

    def splat_par(j, k):
        return plsc.load_gather(par_v, [jnp.full((16,), j * 8 + k, jnp.int32)])

    def roi_body(j, _):
        cx = splat_par(j, 0)
        cy = splat_par(j, 1)
        rw = splat_par(j, 2)
        rh = splat_par(j, 3)
        cs = splat_par(j, 4)
        sn = splat_par(j, 5)
        base = splat_par(j, 6).astype(jnp.int32)

        for c in range(_NCH):
            ay = ay_v[pl.ds(c * 16, 16)]
            ax = ax_v[pl.ds(c * 16, 16)]
            yy = rh * ay
            xx = rw * ax
            x = xx * cs - yy * sn + cx
            y = xx * sn + yy * cs + cy
            valid = ((y > -1.0) & (y < float(_H))
                     & (x > -1.0) & (x < float(_W)))
            xc = jnp.minimum(jnp.maximum(x, 0.0), float(_W - 1))
            yc = jnp.minimum(jnp.maximum(y, 0.0), float(_H - 1))
            x0 = xc.astype(jnp.int32)
            y0 = yc.astype(jnp.int32)
            lx = xc - x0.astype(jnp.float32)
            ly = yc - y0.astype(jnp.float32)
            hx = 1.0 - lx
            hy = 1.0 - ly
            vm = jnp.where(valid, 0.25, 0.0)
            r00 = base + y0 * _W + x0
            half = c // 7
            col = (c % 7) * 16
            ws = (hy * hx * vm, hy * lx * vm, ly * hx * vm, ly * lx * vm)
            rs = (r00, r00 + _W)
            for k in range(2):
                idx_v[2 * k + half, pl.ds(col, 16)] = rs[k]
            for k in range(4):
                w_v[pl.ds(k * _PPTS + c * 16, 16)] = ws[k]

        copies = []
        for k in range(2):
            copies.append(pltpu.async_copy(
                tbl_hbm.at[idx_v.at[2 * k]],
                g_v.at[pl.ds(k * _PTS, 112)], sem))
            copies.append(pltpu.async_copy(
                tbl_hbm.at[idx_v.at[2 * k + 1, pl.ds(0, 84)]],
                g_v.at[pl.ds(k * _PTS + 112, 84)], sem))
        for cp in copies:
            cp.wait()

        def bin_body(b, _):
            accs = None
            for s in range(4):
                for k in range(2):
                    r = k * _PTS + 4 * b + s
                    wx0 = plsc.load_gather(
                        w_v, [jnp.full((16,), 2 * k * _PPTS + 4 * b + s,
                                       jnp.int32)])
                    wx1 = plsc.load_gather(
                        w_v, [jnp.full((16,), (2 * k + 1) * _PPTS + 4 * b + s,
                                       jnp.int32)])
                    terms = []
                    for g8 in range(8):
                        pk = g_v[r, pl.ds(g8 * 16, 16)]
                        w = wx0 if g8 < 4 else wx1
                        ev = lax.bitcast_convert_type(pk << 16, jnp.float32)
                        od = lax.bitcast_convert_type(pk, jnp.float32)
                        terms.append(w * ev)
                        terms.append(w * od)
                    if accs is None:
                        accs = terms
                    else:
                        accs = [a + t for a, t in zip(accs, terms)]
            accs = [accs[i] + accs[i + 8] for i in range(8)]
            for g4 in range(4):
                for par in range(2):
                    sidx = (lanes * 2 + g4 * 32 + par) * (_OH * _OW) + b
                    plsc.store_scatter(out_v, [sidx], accs[g4 * 2 + par])
            return _

        lax.fori_loop(0, _OH * _OW, bin_body, None)
        pltpu.sync_copy(out_v, out_hbm.at[wid * _RPW + j])
        return _

    lax.fori_loop(0, _RPW, roi_body, None)


@jax.jit
def _roi_align_sc(tblf, params, ayc, axc):
    mesh = plsc.VectorSubcoreMesh(core_axis_name="c", subcore_axis_name="s")
    pair = functools.partial(
        pl.kernel,
        out_type=jax.ShapeDtypeStruct((_TBL, _C), jnp.int32),
        mesh=mesh,
        compiler_params=pltpu.CompilerParams(needs_layout_passes=False),
        scratch_types=[
            pltpu.VMEM((136, _C), jnp.float32),
            pltpu.VMEM((_PCHUNK, _C), jnp.int32),
            pltpu.SemaphoreType.DMA,
        ],
    )(_pack_body)(tblf)
    f = functools.partial(
        pl.kernel,
        out_type=jax.ShapeDtypeStruct((_N, _OUTF), jnp.float32),
        mesh=mesh,
        compiler_params=pltpu.CompilerParams(needs_layout_passes=False),
        scratch_types=[
            pltpu.VMEM((_RPW * 8,), jnp.float32),
            pltpu.VMEM((_PPTS,), jnp.float32),
            pltpu.VMEM((_PPTS,), jnp.float32),
            pltpu.VMEM((4, 112), jnp.int32),
            pltpu.VMEM((4 * _PPTS,), jnp.float32),
            pltpu.VMEM((_ROWS, _C), jnp.int32),
            pltpu.VMEM((_OUTF,), jnp.float32),
            pltpu.SemaphoreType.DMA,
        ],
    )(_sc_body)
    return f(pair, params, ayc, axc)


def kernel(inputs, rois):
    tbl = jnp.transpose(inputs, (0, 2, 3, 1)).reshape(_B * _H * _W, _C)
    tbl = jnp.concatenate([tbl, jnp.zeros((_TBLF - _B * _H * _W, _C),
                                          jnp.float32)], axis=0)
    cx = rois[:, 1] * _SCALE
    cy = rois[:, 2] * _SCALE
    rw = jnp.maximum(rois[:, 3] * _SCALE, 1.0)
    rh = jnp.maximum(rois[:, 4] * _SCALE, 1.0)
    cs = jnp.cos(rois[:, 5])
    sn = jnp.sin(rois[:, 5])
    base = rois[:, 0] * float(_H * _W)
    zero = jnp.zeros_like(cx)
    params = jnp.stack([cx, cy, rw, rh, cs, sn, base, zero], 1).reshape(-1)
    out = _roi_align_sc(tbl, params,
                        jnp.asarray(_AY_CONST), jnp.asarray(_AX_CONST))
    return out.reshape(_N, _C, _OH, _OW)


# speedup vs baseline: 1.8070x; 1.0017x over previous
import functools

import numpy as np
import jax
import jax.numpy as jnp
from jax import lax
from jax.experimental import pallas as pl
from jax.experimental.pallas import tpu as pltpu
from jax.experimental.pallas import tpu_sc as plsc

_OH, _OW, _G = 7, 7, 2
_SCALE = 0.25
_B, _C, _H, _W = 2, 128, 200, 200
_N = 1024

_PTS = _OH * _OW * _G * _G
_PPTS = 224
_NCH = _PPTS // 16
_ROWS = 2 * _PTS
_NC, _NS = 2, 16
_NWORK = _NC * _NS
_RPW = _N // _NWORK
_OUTF = _C * _OH * _OW
_CW = _C // 2

_PCHUNK = 128
_CPW = 20
_TBL = _NWORK * _CPW * _PCHUNK
_TBLF = _TBL + 8


def _point_consts():
    ay = np.zeros(_PPTS, np.float32)
    ax = np.zeros(_PPTS, np.float32)
    for p in range(_PTS):
        b, s = divmod(p, _G * _G)
        oh, ow = divmod(b, _OW)
        gy, gx = divmod(s, _G)
        ay[p] = (oh + (gy + 0.5) / _G) / _OH - 0.5
        ax[p] = (ow + (gx + 0.5) / _G) / _OW - 0.5
    ay[_PTS:] = ay[_PTS - 1]
    ax[_PTS:] = ax[_PTS - 1]
    return ay, ax

_AY_CONST, _AX_CONST = _point_consts()


def _pack_body(src_hbm, pair_hbm, in_v, out_v, sem):
    wid = lax.axis_index("s") * _NC + lax.axis_index("c")
    base_row = wid * (_CPW * _PCHUNK)

    def chunk_body(ci, _):
        a = base_row + ci * _PCHUNK
        pltpu.sync_copy(src_hbm.at[pl.ds(a, 136)], in_v)

        def pix_body(jj, _):
            for g in range(4):
                lo = lax.bitcast_convert_type(
                    in_v[jj, pl.ds(g * 32, 16)], jnp.int32)
                hi = lax.bitcast_convert_type(
                    in_v[jj, pl.ds(g * 32 + 16, 16)], jnp.int32)
                w = (lax.shift_right_logical(lo + 32768, 16)
                     | ((hi + 32768) & jnp.int32(-65536)))

                @pl.when(jj < _PCHUNK)
                def _store_lo():
                    out_v[jj, pl.ds(g * 16, 16)] = w

                @pl.when(jj > 0)
                def _store_hi():
                    out_v[jj - 1, pl.ds(_CW + g * 16, 16)] = w
            return _

        lax.fori_loop(0, _PCHUNK + 1, pix_body, None)
        pltpu.sync_copy(out_v, pair_hbm.at[pl.ds(a, _PCHUNK)])
        return _

    lax.fori_loop(0, _CPW, chunk_body, None)


def _sc_body(tbl_hbm, par_hbm, ay_hbm, ax_hbm, out_hbm,
             par_v, ay_v, ax_v, idx_v, w_v, g_v, out_v, sem):
    wid = lax.axis_index("s") * _NC + lax.axis_index("c")
    pltpu.sync_copy(ay_hbm, ay_v)
    pltpu.sync_copy(ax_hbm, ax_v)
    pltpu.sync_copy(par_hbm.at[pl.ds(wid * (_RPW * 8), _RPW * 8)], par_v)

    lanes = lax.iota(jnp.int32, 16)

    def splat_par(j, k):
        return plsc.load_gather(par_v, [jnp.full((16,), j * 8 + k, jnp.int32)])

    def roi_body(j, _):
        cx = splat_par(j, 0)
        cy = splat_par(j, 1)
        rw = splat_par(j, 2)
        rh = splat_par(j, 3)
        cs = splat_par(j, 4)
        sn = splat_par(j, 5)
        base = splat_par(j, 6).astype(jnp.int32)

        for c in range(_NCH):
            ay = ay_v[pl.ds(c * 16, 16)]
            ax = ax_v[pl.ds(c * 16, 16)]
            yy = rh * ay
            xx = rw * ax
            x = xx * cs - yy * sn + cx
            y = xx * sn + yy * cs + cy
            valid = ((y > -1.0) & (y < float(_H))
                     & (x > -1.0) & (x < float(_W)))
            xc = jnp.minimum(jnp.maximum(x, 0.0), float(_W - 1))
            yc = jnp.minimum(jnp.maximum(y, 0.0), float(_H - 1))
            x0 = xc.astype(jnp.int32)
            y0 = yc.astype(jnp.int32)
            lx = xc - x0.astype(jnp.float32)
            ly = yc - y0.astype(jnp.float32)
            hx = 1.0 - lx
            hy = 1.0 - ly
            vm = jnp.where(valid, 0.25, 0.0)
            r00 = base + y0 * _W + x0
            half = c // 7
            col = (c % 7) * 16
            ws = (hy * hx * vm, hy * lx * vm, ly * hx * vm, ly * lx * vm)
            rs = (r00, r00 + _W)
            for k in range(2):
                idx_v[2 * k + half, pl.ds(col, 16)] = rs[k]
            for k in range(4):
                w_v[pl.ds(k * _PPTS + c * 16, 16)] = ws[k]

        copies = []
        for k in range(2):
            copies.append(pltpu.async_copy(
                tbl_hbm.at[idx_v.at[2 * k]],
                g_v.at[pl.ds(k * _PTS, 112)], sem))
            copies.append(pltpu.async_copy(
                tbl_hbm.at[idx_v.at[2 * k + 1, pl.ds(0, 84)]],
                g_v.at[pl.ds(k * _PTS + 112, 84)], sem))
        for cp in copies:
            cp.wait()

        def bin_body(b, _):
            accs = None
            for s in range(4):
                for k in range(2):
                    r = k * _PTS + 4 * b + s
                    wx0 = plsc.load_gather(
                        w_v, [jnp.full((16,), 2 * k * _PPTS + 4 * b + s,
                                       jnp.int32)])
                    wx1 = plsc.load_gather(
                        w_v, [jnp.full((16,), (2 * k + 1) * _PPTS + 4 * b + s,
                                       jnp.int32)])
                    terms = []
                    for g8 in range(8):
                        pk = g_v[r, pl.ds(g8 * 16, 16)]
                        w = wx0 if g8 < 4 else wx1
                        ev = lax.bitcast_convert_type(pk << 16, jnp.float32)
                        od = lax.bitcast_convert_type(pk, jnp.float32)
                        terms.append(w * ev)
                        terms.append(w * od)
                    if accs is None:
                        accs = terms
                    else:
                        accs = [a + t for a, t in zip(accs, terms)]
            accs = [accs[i] + accs[i + 8] for i in range(8)]
            for g4 in range(4):
                for par in range(2):
                    sidx = (lanes + g4 * 32 + par * 16) * (_OH * _OW) + b
                    plsc.store_scatter(out_v, [sidx], accs[g4 * 2 + par])
            return _

        lax.fori_loop(0, _OH * _OW, bin_body, None)
        pltpu.sync_copy(out_v, out_hbm.at[wid * _RPW + j])
        return _

    lax.fori_loop(0, _RPW, roi_body, None)


@jax.jit
def _roi_align_sc(tblf, params, ayc, axc):
    mesh = plsc.VectorSubcoreMesh(core_axis_name="c", subcore_axis_name="s")
    pair = functools.partial(
        pl.kernel,
        out_type=jax.ShapeDtypeStruct((_TBL, _C), jnp.int32),
        mesh=mesh,
        compiler_params=pltpu.CompilerParams(needs_layout_passes=False),
        scratch_types=[
            pltpu.VMEM((136, _C), jnp.float32),
            pltpu.VMEM((_PCHUNK, _C), jnp.int32),
            pltpu.SemaphoreType.DMA,
        ],
    )(_pack_body)(tblf)
    f = functools.partial(
        pl.kernel,
        out_type=jax.ShapeDtypeStruct((_N, _OUTF), jnp.float32),
        mesh=mesh,
        compiler_params=pltpu.CompilerParams(needs_layout_passes=False),
        scratch_types=[
            pltpu.VMEM((_RPW * 8,), jnp.float32),
            pltpu.VMEM((_PPTS,), jnp.float32),
            pltpu.VMEM((_PPTS,), jnp.float32),
            pltpu.VMEM((4, 112), jnp.int32),
            pltpu.VMEM((4 * _PPTS,), jnp.float32),
            pltpu.VMEM((_ROWS, _C), jnp.int32),
            pltpu.VMEM((_OUTF,), jnp.float32),
            pltpu.SemaphoreType.DMA,
        ],
    )(_sc_body)
    return f(pair, params, ayc, axc)


def kernel(inputs, rois):
    tbl = jnp.transpose(inputs, (0, 2, 3, 1)).reshape(_B * _H * _W, _C)
    tbl = jnp.concatenate([tbl, jnp.zeros((_TBLF - _B * _H * _W, _C),
                                          jnp.float32)], axis=0)
    cx = rois[:, 1] * _SCALE
    cy = rois[:, 2] * _SCALE
    rw = jnp.maximum(rois[:, 3] * _SCALE, 1.0)
    rh = jnp.maximum(rois[:, 4] * _SCALE, 1.0)
    cs = jnp.cos(rois[:, 5])
    sn = jnp.sin(rois[:, 5])
    base = rois[:, 0] * float(_H * _W)
    zero = jnp.zeros_like(cx)
    params = jnp.stack([cx, cy, rw, rh, cs, sn, base, zero], 1).reshape(-1)
    out = _roi_align_sc(tbl, params,
                        jnp.asarray(_AY_CONST), jnp.asarray(_AX_CONST))
    return out.reshape(_N, _C, _OH, _OW)


# speedup vs baseline: 2.3095x; 1.2781x over previous
import functools

import numpy as np
import jax
import jax.numpy as jnp
from jax import lax
from jax.experimental import pallas as pl
from jax.experimental.pallas import tpu as pltpu
from jax.experimental.pallas import tpu_sc as plsc

_OH, _OW, _G = 7, 7, 2
_SCALE = 0.25
_B, _C, _H, _W = 2, 128, 200, 200
_N = 1024

_PTS = _OH * _OW * _G * _G
_PPTS = 224
_NCH = _PPTS // 16
_ROWS = 2 * _PTS
_NC, _NS = 2, 16
_NWORK = _NC * _NS
_RPW = _N // _NWORK
_OUTF = _C * _OH * _OW
_CW = _C // 2

_PCHUNK = 160
_CPW = 16
_TBL = _NWORK * _CPW * _PCHUNK
_NPIX = _B * _H * _W


def _point_consts():
    ay = np.zeros(_PPTS, np.float32)
    ax = np.zeros(_PPTS, np.float32)
    for p in range(_PTS):
        b, s = divmod(p, _G * _G)
        oh, ow = divmod(b, _OW)
        gy, gx = divmod(s, _G)
        ay[p] = (oh + (gy + 0.5) / _G) / _OH - 0.5
        ax[p] = (ow + (gx + 0.5) / _G) / _OW - 0.5
    ay[_PTS:] = ay[_PTS - 1]
    ax[_PTS:] = ax[_PTS - 1]
    return ay, ax

_AY_CONST, _AX_CONST = _point_consts()


def _pack_body(src_hbm, pair_hbm, in0_v, in1_v, out0_v, out1_v,
               sem_in0, sem_in1, sem_out0, sem_out1):
    wid = lax.axis_index("s") * _NC + lax.axis_index("c")
    base_row = wid * (_CPW * _PCHUNK)

    def in_copies(ci, in_v, sem):
        a = base_row + ci * _PCHUNK
        o1 = jnp.minimum(a, _NPIX - _PCHUNK)
        o2 = jnp.minimum(a + _PCHUNK, _NPIX - 8)
        return (pltpu.make_async_copy(src_hbm.at[pl.ds(o1, _PCHUNK)],
                                      in_v.at[pl.ds(0, _PCHUNK)], sem),
                pltpu.make_async_copy(src_hbm.at[pl.ds(o2, 8)],
                                      in_v.at[pl.ds(_PCHUNK, 8)], sem))

    def fire_in(ci, in_v, sem):
        for cp in in_copies(ci, in_v, sem):
            cp.start()

    def wait_in(in_v, sem):
        for cp in in_copies(0, in_v, sem):
            cp.wait()

    def out_copy(ci, out_v, sem):
        a = base_row + ci * _PCHUNK
        return pltpu.make_async_copy(out_v, pair_hbm.at[pl.ds(a, _PCHUNK)],
                                     sem)

    def pack(in_v, out_v):
        def word(jj, g):
            lo = lax.bitcast_convert_type(
                in_v[jj, pl.ds(g * 32, 16)], jnp.int32)
            hi = lax.bitcast_convert_type(
                in_v[jj, pl.ds(g * 32 + 16, 16)], jnp.int32)
            return (lax.shift_right_logical(lo + 32768, 16)
                    | ((hi + 32768) & jnp.int32(-65536)))

        for g in range(4):
            out_v[0, pl.ds(g * 16, 16)] = word(0, g)
            w_last = word(_PCHUNK, g)
            out_v[_PCHUNK - 1, pl.ds(_CW + g * 16, 16)] = w_last

        def pix_body(jj, _):
            for g in range(4):
                w = word(jj, g)
                out_v[jj, pl.ds(g * 16, 16)] = w
                out_v[jj - 1, pl.ds(_CW + g * 16, 16)] = w
            return _

        lax.fori_loop(1, _PCHUNK, pix_body, None)

    fire_in(0, in0_v, sem_in0)

    def chunk_pair(i, _):
        c0 = 2 * i
        wait_in(in0_v, sem_in0)
        fire_in(c0 + 1, in1_v, sem_in1)

        @pl.when(i > 0)
        def _drain_out0():
            out_copy(0, out0_v, sem_out0).wait()

        pack(in0_v, out0_v)
        out_copy(c0, out0_v, sem_out0).start()

        wait_in(in1_v, sem_in1)
        fire_in(jnp.minimum(c0 + 2, _CPW - 1), in0_v, sem_in0)

        @pl.when(i > 0)
        def _drain_out1():
            out_copy(0, out1_v, sem_out1).wait()

        pack(in1_v, out1_v)
        out_copy(c0 + 1, out1_v, sem_out1).start()
        return _

    lax.fori_loop(0, _CPW // 2, chunk_pair, None)
    wait_in(in0_v, sem_in0)
    out_copy(0, out0_v, sem_out0).wait()
    out_copy(0, out1_v, sem_out1).wait()


def _sc_body(tbl_hbm, par_hbm, ay_hbm, ax_hbm, out_hbm,
             par_v, ay_v, ax_v, idx_v, w_v, g_v, out_v, sem):
    wid = lax.axis_index("s") * _NC + lax.axis_index("c")
    pltpu.sync_copy(ay_hbm, ay_v)
    pltpu.sync_copy(ax_hbm, ax_v)
    pltpu.sync_copy(par_hbm.at[pl.ds(wid * (_RPW * 8), _RPW * 8)], par_v)

    lanes = lax.iota(jnp.int32, 16)

    def splat_par(j, k):
        return plsc.load_gather(par_v, [jnp.full((16,), j * 8 + k, jnp.int32)])

    def roi_body(j, _):
        cx = splat_par(j, 0)
        cy = splat_par(j, 1)
        rw = splat_par(j, 2)
        rh = splat_par(j, 3)
        cs = splat_par(j, 4)
        sn = splat_par(j, 5)
        base = splat_par(j, 6).astype(jnp.int32)

        for c in range(_NCH):
            ay = ay_v[pl.ds(c * 16, 16)]
            ax = ax_v[pl.ds(c * 16, 16)]
            yy = rh * ay
            xx = rw * ax
            x = xx * cs - yy * sn + cx
            y = xx * sn + yy * cs + cy
            valid = ((y > -1.0) & (y < float(_H))
                     & (x > -1.0) & (x < float(_W)))
            xc = jnp.minimum(jnp.maximum(x, 0.0), float(_W - 1))
            yc = jnp.minimum(jnp.maximum(y, 0.0), float(_H - 1))
            x0 = xc.astype(jnp.int32)
            y0 = yc.astype(jnp.int32)
            lx = xc - x0.astype(jnp.float32)
            ly = yc - y0.astype(jnp.float32)
            hx = 1.0 - lx
            hy = 1.0 - ly
            vm = jnp.where(valid, 0.25, 0.0)
            r00 = base + y0 * _W + x0
            half = c // 7
            col = (c % 7) * 16
            ws = (hy * hx * vm, hy * lx * vm, ly * hx * vm, ly * lx * vm)
            rs = (r00, r00 + _W)
            for k in range(2):
                idx_v[2 * k + half, pl.ds(col, 16)] = rs[k]
            for k in range(4):
                w_v[pl.ds(k * _PPTS + c * 16, 16)] = ws[k]

        copies = []
        for k in range(2):
            copies.append(pltpu.async_copy(
                tbl_hbm.at[idx_v.at[2 * k]],
                g_v.at[pl.ds(k * _PTS, 112)], sem))
            copies.append(pltpu.async_copy(
                tbl_hbm.at[idx_v.at[2 * k + 1, pl.ds(0, 84)]],
                g_v.at[pl.ds(k * _PTS + 112, 84)], sem))
        for cp in copies:
            cp.wait()

        def bin_body(b, _):
            accs = None
            for s in range(4):
                for k in range(2):
                    r = k * _PTS + 4 * b + s
                    wx0 = plsc.load_gather(
                        w_v, [jnp.full((16,), 2 * k * _PPTS + 4 * b + s,
                                       jnp.int32)])
                    wx1 = plsc.load_gather(
                        w_v, [jnp.full((16,), (2 * k + 1) * _PPTS + 4 * b + s,
                                       jnp.int32)])
                    terms = []
                    for g8 in range(8):
                        pk = g_v[r, pl.ds(g8 * 16, 16)]
                        w = wx0 if g8 < 4 else wx1
                        ev = lax.bitcast_convert_type(pk << 16, jnp.float32)
                        od = lax.bitcast_convert_type(pk, jnp.float32)
                        terms.append(w * ev)
                        terms.append(w * od)
                    if accs is None:
                        accs = terms
                    else:
                        accs = [a + t for a, t in zip(accs, terms)]
            accs = [accs[i] + accs[i + 8] for i in range(8)]
            for g4 in range(4):
                for par in range(2):
                    sidx = (lanes + g4 * 32 + par * 16) * (_OH * _OW) + b
                    plsc.store_scatter(out_v, [sidx], accs[g4 * 2 + par])
            return _

        lax.fori_loop(0, _OH * _OW, bin_body, None)
        pltpu.sync_copy(out_v, out_hbm.at[wid * _RPW + j])
        return _

    lax.fori_loop(0, _RPW, roi_body, None)


@jax.jit
def _roi_align_sc(tblf, params, ayc, axc):
    mesh = plsc.VectorSubcoreMesh(core_axis_name="c", subcore_axis_name="s")
    pair = functools.partial(
        pl.kernel,
        out_type=jax.ShapeDtypeStruct((_TBL, _C), jnp.int32),
        mesh=mesh,
        compiler_params=pltpu.CompilerParams(needs_layout_passes=False),
        scratch_types=[
            pltpu.VMEM((_PCHUNK + 8, _C), jnp.float32),
            pltpu.VMEM((_PCHUNK + 8, _C), jnp.float32),
            pltpu.VMEM((_PCHUNK, _C), jnp.int32),
            pltpu.VMEM((_PCHUNK, _C), jnp.int32),
            pltpu.SemaphoreType.DMA,
            pltpu.SemaphoreType.DMA,
            pltpu.SemaphoreType.DMA,
            pltpu.SemaphoreType.DMA,
        ],
    )(_pack_body)(tblf)
    f = functools.partial(
        pl.kernel,
        out_type=jax.ShapeDtypeStruct((_N, _OUTF), jnp.float32),
        mesh=mesh,
        compiler_params=pltpu.CompilerParams(needs_layout_passes=False),
        scratch_types=[
            pltpu.VMEM((_RPW * 8,), jnp.float32),
            pltpu.VMEM((_PPTS,), jnp.float32),
            pltpu.VMEM((_PPTS,), jnp.float32),
            pltpu.VMEM((4, 112), jnp.int32),
            pltpu.VMEM((4 * _PPTS,), jnp.float32),
            pltpu.VMEM((_ROWS, _C), jnp.int32),
            pltpu.VMEM((_OUTF,), jnp.float32),
            pltpu.SemaphoreType.DMA,
        ],
    )(_sc_body)
    return f(pair, params, ayc, axc)


def kernel(inputs, rois):
    tbl = jnp.transpose(inputs, (0, 2, 3, 1)).reshape(_NPIX, _C)
    cx = rois[:, 1] * _SCALE
    cy = rois[:, 2] * _SCALE
    rw = jnp.maximum(rois[:, 3] * _SCALE, 1.0)
    rh = jnp.maximum(rois[:, 4] * _SCALE, 1.0)
    cs = jnp.cos(rois[:, 5])
    sn = jnp.sin(rois[:, 5])
    base = rois[:, 0] * float(_H * _W)
    zero = jnp.zeros_like(cx)
    params = jnp.stack([cx, cy, rw, rh, cs, sn, base, zero], 1).reshape(-1)
    out = _roi_align_sc(tbl, params,
                        jnp.asarray(_AY_CONST), jnp.asarray(_AX_CONST))
    return out.reshape(_N, _C, _OH, _OW)


# speedup vs baseline: 3.1070x; 1.3453x over previous
import functools

import numpy as np
import jax
import jax.numpy as jnp
from jax import lax
from jax.experimental import pallas as pl
from jax.experimental.pallas import tpu as pltpu
from jax.experimental.pallas import tpu_sc as plsc

_OH, _OW, _G = 7, 7, 2
_SCALE = 0.25
_B, _C, _H, _W = 2, 128, 200, 200
_N = 1024

_PTS = _OH * _OW * _G * _G
_PPTS = 224
_NCH = _PPTS // 16
_ROWS = 2 * _PTS
_NC, _NS = 2, 16
_NWORK = _NC * _NS
_RPW = _N // _NWORK
_OUTF = _C * _OH * _OW
_CW = _C // 2

_PCHUNK = 160
_CPW = 16
_TBL = _NWORK * _CPW * _PCHUNK
_NPIX = _B * _H * _W


def _point_consts():
    ay = np.zeros(_PPTS, np.float32)
    ax = np.zeros(_PPTS, np.float32)
    for p in range(_PTS):
        b, s = divmod(p, _G * _G)
        oh, ow = divmod(b, _OW)
        gy, gx = divmod(s, _G)
        ay[p] = (oh + (gy + 0.5) / _G) / _OH - 0.5
        ax[p] = (ow + (gx + 0.5) / _G) / _OW - 0.5
    ay[_PTS:] = ay[_PTS - 1]
    ax[_PTS:] = ax[_PTS - 1]
    return ay, ax

_AY_CONST, _AX_CONST = _point_consts()


def _pack_body(src_hbm, pair_hbm, in0_v, in1_v, out0_v, out1_v,
               sem_in0, sem_in1, sem_out0, sem_out1):
    wid = lax.axis_index("s") * _NC + lax.axis_index("c")
    base_row = wid * (_CPW * _PCHUNK)

    def in_copies(ci, in_v, sem):
        a = base_row + ci * _PCHUNK
        o1 = jnp.minimum(a, _NPIX - _PCHUNK)
        o2 = jnp.minimum(a + _PCHUNK, _NPIX - 8)
        return (pltpu.make_async_copy(src_hbm.at[pl.ds(o1, _PCHUNK)],
                                      in_v.at[pl.ds(0, _PCHUNK)], sem),
                pltpu.make_async_copy(src_hbm.at[pl.ds(o2, 8)],
                                      in_v.at[pl.ds(_PCHUNK, 8)], sem))

    def fire_in(ci, in_v, sem):
        for cp in in_copies(ci, in_v, sem):
            cp.start()

    def wait_in(in_v, sem):
        for cp in in_copies(0, in_v, sem):
            cp.wait()

    def out_copy(ci, out_v, sem):
        a = base_row + ci * _PCHUNK
        return pltpu.make_async_copy(out_v, pair_hbm.at[pl.ds(a, _PCHUNK)],
                                     sem)

    def pack(in_v, out_v):
        def word(jj, g):
            lo = lax.bitcast_convert_type(
                in_v[jj, pl.ds(g * 32, 16)], jnp.int32)
            hi = lax.bitcast_convert_type(
                in_v[jj, pl.ds(g * 32 + 16, 16)], jnp.int32)
            return (lax.shift_right_logical(lo + 32768, 16)
                    | ((hi + 32768) & jnp.int32(-65536)))

        for g in range(4):
            out_v[0, pl.ds(g * 16, 16)] = word(0, g)
            w_last = word(_PCHUNK, g)
            out_v[_PCHUNK - 1, pl.ds(_CW + g * 16, 16)] = w_last

        def pix_body(jj, _):
            for g in range(4):
                w = word(jj, g)
                out_v[jj, pl.ds(g * 16, 16)] = w
                out_v[jj - 1, pl.ds(_CW + g * 16, 16)] = w
            return _

        lax.fori_loop(1, _PCHUNK, pix_body, None)

    fire_in(0, in0_v, sem_in0)

    def chunk_pair(i, _):
        c0 = 2 * i
        wait_in(in0_v, sem_in0)
        fire_in(c0 + 1, in1_v, sem_in1)

        @pl.when(i > 0)
        def _drain_out0():
            out_copy(0, out0_v, sem_out0).wait()

        pack(in0_v, out0_v)
        out_copy(c0, out0_v, sem_out0).start()

        wait_in(in1_v, sem_in1)
        fire_in(jnp.minimum(c0 + 2, _CPW - 1), in0_v, sem_in0)

        @pl.when(i > 0)
        def _drain_out1():
            out_copy(0, out1_v, sem_out1).wait()

        pack(in1_v, out1_v)
        out_copy(c0 + 1, out1_v, sem_out1).start()
        return _

    lax.fori_loop(0, _CPW // 2, chunk_pair, None)
    wait_in(in0_v, sem_in0)
    out_copy(0, out0_v, sem_out0).wait()
    out_copy(0, out1_v, sem_out1).wait()


def _sc_body(tbl_hbm, par_hbm, ay_hbm, ax_hbm, out_hbm,
             par_v, ay_v, ax_v, idx0_v, idx1_v, w0_v, w1_v, g0_v, g1_v,
             o0_v, o1_v, sg0, sg1, so0, so1):
    wid = lax.axis_index("s") * _NC + lax.axis_index("c")
    pltpu.sync_copy(ay_hbm, ay_v)
    pltpu.sync_copy(ax_hbm, ax_v)
    pltpu.sync_copy(par_hbm.at[pl.ds(wid * (_RPW * 8), _RPW * 8)], par_v)

    lanes = lax.iota(jnp.int32, 16)

    def splat_par(j, k):
        return plsc.load_gather(par_v, [jnp.full((16,), j * 8 + k, jnp.int32)])

    def coords(j, idx_v, w_v):
        cx = splat_par(j, 0)
        cy = splat_par(j, 1)
        rw = splat_par(j, 2)
        rh = splat_par(j, 3)
        cs = splat_par(j, 4)
        sn = splat_par(j, 5)
        base = splat_par(j, 6).astype(jnp.int32)
        for c in range(_NCH):
            ay = ay_v[pl.ds(c * 16, 16)]
            ax = ax_v[pl.ds(c * 16, 16)]
            yy = rh * ay
            xx = rw * ax
            x = xx * cs - yy * sn + cx
            y = xx * sn + yy * cs + cy
            valid = ((y > -1.0) & (y < float(_H))
                     & (x > -1.0) & (x < float(_W)))
            xc = jnp.minimum(jnp.maximum(x, 0.0), float(_W - 1))
            yc = jnp.minimum(jnp.maximum(y, 0.0), float(_H - 1))
            x0 = xc.astype(jnp.int32)
            y0 = yc.astype(jnp.int32)
            lx = xc - x0.astype(jnp.float32)
            ly = yc - y0.astype(jnp.float32)
            hx = 1.0 - lx
            hy = 1.0 - ly
            vm = jnp.where(valid, 0.25, 0.0)
            r00 = base + y0 * _W + x0
            half = c // 7
            col = (c % 7) * 16
            ws = (hy * hx * vm, hy * lx * vm, ly * hx * vm, ly * lx * vm)
            rs = (r00, r00 + _W)
            for k in range(2):
                idx_v[2 * k + half, pl.ds(col, 16)] = rs[k]
            for k in range(4):
                w_v[pl.ds(k * _PPTS + c * 16, 16)] = ws[k]

    def g_copies(idx_v, g_v, sem):
        cps = []
        for k in range(2):
            cps.append(pltpu.make_async_copy(
                tbl_hbm.at[idx_v.at[2 * k]],
                g_v.at[pl.ds(k * _PTS, 112)], sem))
            cps.append(pltpu.make_async_copy(
                tbl_hbm.at[idx_v.at[2 * k + 1, pl.ds(0, 84)]],
                g_v.at[pl.ds(k * _PTS + 112, 84)], sem))
        return cps

    def fire_g(idx_v, g_v, sem):
        for cp in g_copies(idx_v, g_v, sem):
            cp.start()

    def wait_g(idx_v, g_v, sem):
        for cp in g_copies(idx_v, g_v, sem):
            cp.wait()

    def out_cp(j, out_v, sem):
        return pltpu.make_async_copy(out_v, out_hbm.at[wid * _RPW + j], sem)

    def accumulate(g_v, w_v, out_v):
        def bin_body(b, _):
            accs = None
            for s in range(4):
                for k in range(2):
                    r = k * _PTS + 4 * b + s
                    wx0 = plsc.load_gather(
                        w_v, [jnp.full((16,), 2 * k * _PPTS + 4 * b + s,
                                       jnp.int32)])
                    wx1 = plsc.load_gather(
                        w_v, [jnp.full((16,), (2 * k + 1) * _PPTS + 4 * b + s,
                                       jnp.int32)])
                    terms = []
                    for g8 in range(8):
                        pk = g_v[r, pl.ds(g8 * 16, 16)]
                        w = wx0 if g8 < 4 else wx1
                        ev = lax.bitcast_convert_type(pk << 16, jnp.float32)
                        od = lax.bitcast_convert_type(pk, jnp.float32)
                        terms.append(w * ev)
                        terms.append(w * od)
                    if accs is None:
                        accs = terms
                    else:
                        accs = [a + t for a, t in zip(accs, terms)]
            accs = [accs[i] + accs[i + 8] for i in range(8)]
            for g4 in range(4):
                for par in range(2):
                    sidx = (lanes + g4 * 32 + par * 16) * (_OH * _OW) + b
                    plsc.store_scatter(out_v, [sidx], accs[g4 * 2 + par])
            return _

        lax.fori_loop(0, _OH * _OW, bin_body, None)

    coords(0, idx0_v, w0_v)
    fire_g(idx0_v, g0_v, sg0)

    def pair_body(i, _):
        j0 = 2 * i
        wait_g(idx0_v, g0_v, sg0)
        coords(j0 + 1, idx1_v, w1_v)
        fire_g(idx1_v, g1_v, sg1)

        @pl.when(i > 0)
        def _drain_o0():
            out_cp(0, o0_v, so0).wait()

        accumulate(g0_v, w0_v, o0_v)
        out_cp(j0, o0_v, so0).start()

        wait_g(idx1_v, g1_v, sg1)
        coords(jnp.minimum(j0 + 2, _RPW - 1), idx0_v, w0_v)
        fire_g(idx0_v, g0_v, sg0)

        @pl.when(i > 0)
        def _drain_o1():
            out_cp(0, o1_v, so1).wait()

        accumulate(g1_v, w1_v, o1_v)
        out_cp(j0 + 1, o1_v, so1).start()
        return _

    lax.fori_loop(0, _RPW // 2, pair_body, None)
    wait_g(idx0_v, g0_v, sg0)
    out_cp(0, o0_v, so0).wait()
    out_cp(0, o1_v, so1).wait()


@jax.jit
def _roi_align_sc(tblf, params, ayc, axc):
    mesh = plsc.VectorSubcoreMesh(core_axis_name="c", subcore_axis_name="s")
    pair = functools.partial(
        pl.kernel,
        out_type=jax.ShapeDtypeStruct((_TBL, _C), jnp.int32),
        mesh=mesh,
        compiler_params=pltpu.CompilerParams(needs_layout_passes=False),
        scratch_types=[
            pltpu.VMEM((_PCHUNK + 8, _C), jnp.float32),
            pltpu.VMEM((_PCHUNK + 8, _C), jnp.float32),
            pltpu.VMEM((_PCHUNK, _C), jnp.int32),
            pltpu.VMEM((_PCHUNK, _C), jnp.int32),
            pltpu.SemaphoreType.DMA,
            pltpu.SemaphoreType.DMA,
            pltpu.SemaphoreType.DMA,
            pltpu.SemaphoreType.DMA,
        ],
    )(_pack_body)(tblf)
    f = functools.partial(
        pl.kernel,
        out_type=jax.ShapeDtypeStruct((_N, _OUTF), jnp.float32),
        mesh=mesh,
        compiler_params=pltpu.CompilerParams(needs_layout_passes=False),
        scratch_types=[
            pltpu.VMEM((_RPW * 8,), jnp.float32),
            pltpu.VMEM((_PPTS,), jnp.float32),
            pltpu.VMEM((_PPTS,), jnp.float32),
            pltpu.VMEM((4, 112), jnp.int32),
            pltpu.VMEM((4, 112), jnp.int32),
            pltpu.VMEM((4 * _PPTS,), jnp.float32),
            pltpu.VMEM((4 * _PPTS,), jnp.float32),
            pltpu.VMEM((_ROWS, _C), jnp.int32),
            pltpu.VMEM((_ROWS, _C), jnp.int32),
            pltpu.VMEM((_OUTF,), jnp.float32),
            pltpu.VMEM((_OUTF,), jnp.float32),
            pltpu.SemaphoreType.DMA,
            pltpu.SemaphoreType.DMA,
            pltpu.SemaphoreType.DMA,
            pltpu.SemaphoreType.DMA,
        ],
    )(_sc_body)
    return f(pair, params, ayc, axc)


def kernel(inputs, rois):
    tbl = jnp.transpose(inputs, (0, 2, 3, 1)).reshape(_NPIX, _C)
    cx = rois[:, 1] * _SCALE
    cy = rois[:, 2] * _SCALE
    rw = jnp.maximum(rois[:, 3] * _SCALE, 1.0)
    rh = jnp.maximum(rois[:, 4] * _SCALE, 1.0)
    cs = jnp.cos(rois[:, 5])
    sn = jnp.sin(rois[:, 5])
    base = rois[:, 0] * float(_H * _W)
    zero = jnp.zeros_like(cx)
    params = jnp.stack([cx, cy, rw, rh, cs, sn, base, zero], 1).reshape(-1)
    out = _roi_align_sc(tbl, params,
                        jnp.asarray(_AY_CONST), jnp.asarray(_AX_CONST))
    return out.reshape(_N, _C, _OH, _OW)
